# baseline (speedup 1.0000x reference)
import jax
import jax.numpy as jnp
from jax import lax
from jax.experimental import pallas as pl
from jax.experimental.pallas import tpu as pltpu
from jax.experimental.pallas import tpu_sc as plsc

N = 10000
E = 320000
D = 128
H = 128
Z = 64
DH = 256
OUT = 231
G = 64

NC = 2
NS = 16
NW = NC * NS
CH = 128
NCH = E // CH
CPW = 80
NPA = 10240
RPS = NPA // NS
NPD = 10240
BR = 512
NBLK = NPA // BR

_mesh = plsc.VectorSubcoreMesh(core_axis_name="c", subcore_axis_name="s")


def _zero_rows(buf, nrows):
    z = jnp.zeros((16,), jnp.float32)

    @pl.loop(0, nrows)
    def _(r):
        for k in range(8):
            buf[r, pl.ds(k * 16, 16)] = z


_OUT_CHUNKS = [(0, 128), (128, 128), (256, 128), (384, 128), (512, 128)]


def _stage_dst_table(ei_hbm, didx_t, wid, semi, wait):
    last = NW - 1

    @pl.when(wid < last)
    def _():
        off = pl.multiple_of(wid * CPW, 8)
        if wait:
            pltpu.make_async_copy(ei_hbm.at[1, pl.ds(off, CPW)], didx_t,
                                  semi).wait()
        else:
            pltpu.async_copy(ei_hbm.at[1, pl.ds(off, CPW)], didx_t, semi)

    @pl.when(wid == last)
    def _():
        ops = [(ei_hbm.at[1, pl.ds(NCH - 20, 16)], didx_t.at[pl.ds(0, 16)])]
        ops += [(ei_hbm.at[1, NCH - 4 + k], didx_t.at[16 + k])
                for k in range(4)]
        for src, dst in ops:
            if wait:
                pltpu.make_async_copy(src, dst, semi).wait()
            else:
                pltpu.async_copy(src, dst, semi)


def _deg_body(ei_hbm, deg_out, didx_t, ones_v, zrow_v, deg_sp, semi):
    c = lax.axis_index("c")
    s = lax.axis_index("s")
    wid = s * NC + c
    cb = jnp.minimum(wid * CPW, NCH - 20)
    nch = jnp.where(wid == NW - 1, 20, CPW)

    _stage_dst_table(ei_hbm, didx_t, wid, semi, wait=False)

    @pl.loop(0, NPD // NS // 16)
    def _(r):
        zrow_v[pl.ds(r * 16, 16)] = jnp.zeros((16,), jnp.float32)
    for k in range(8):
        ones_v[pl.ds(k * 16, 16)] = jnp.ones((16,), jnp.float32)
    pltpu.sync_copy(zrow_v,
                    deg_sp.at[pl.ds(pl.multiple_of(s * (NPD // NS), 8),
                                    NPD // NS)])
    _stage_dst_table(ei_hbm, didx_t, wid, semi, wait=True)
    plsc.subcore_barrier()

    @pl.loop(0, nch)
    def _(j):
        pltpu.sync_copy(ones_v, deg_sp.at[didx_t.at[j]], add=True)

    plsc.subcore_barrier()
    pltpu.sync_copy(
        deg_sp.at[pl.ds(pl.multiple_of(s * (NPD // NS), 8), NPD // NS)],
        zrow_v)
    pltpu.sync_copy(
        zrow_v,
        deg_out.at[c, pl.ds(pl.multiple_of(s * (NPD // NS), 8), NPD // NS)])


_deg_call = pl.kernel(
    _deg_body,
    out_type=jax.ShapeDtypeStruct((NC, NPD), jnp.float32),
    mesh=_mesh,
    scratch_types=[
        pltpu.VMEM((CPW, CH), jnp.int32),
        pltpu.VMEM((CH,), jnp.float32),
        pltpu.VMEM((NPD // NS,), jnp.float32),
        pltpu.VMEM_SHARED((NPD,), jnp.float32),
        pltpu.SemaphoreType.DMA,
    ],
)


def _spmm_body(h_hbm, ei_hbm, out_hbm,
               sidx_a, sidx_b, didx_t, rows_a, rows_b, acc_sp,
               semg_a, semg_b, sems_a, sems_b, semi):
    c = lax.axis_index("c")
    s = lax.axis_index("s")
    wid = s * NC + c
    cb = jnp.minimum(wid * CPW, NCH - 20)
    npair = jnp.where(wid == NW - 1, 10, CPW // 2)

    _stage_dst_table(ei_hbm, didx_t, wid, semi, wait=False)
    pltpu.sync_copy(ei_hbm.at[0, cb], sidx_a)
    pltpu.async_copy(h_hbm.at[sidx_a], rows_a, semg_a)

    _zero_rows(rows_b, CH)
    for (r0, rn) in _OUT_CHUNKS:
        off = pl.multiple_of(s * RPS + r0, 8)
        pltpu.async_copy(rows_b, acc_sp.at[pl.ds(off, rn)], sems_b)
    for (r0, rn) in _OUT_CHUNKS:
        off = pl.multiple_of(s * RPS + r0, 8)
        pltpu.make_async_copy(rows_b, acc_sp.at[pl.ds(off, rn)],
                              sems_b).wait()
    _stage_dst_table(ei_hbm, didx_t, wid, semi, wait=True)
    plsc.subcore_barrier()

    @pl.loop(0, npair)
    def _(jj):
        j = jj * 2
        pltpu.sync_copy(ei_hbm.at[0, cb + j + 1], sidx_b)

        @pl.when(jj > 0)
        def _():
            pltpu.make_async_copy(rows_b, acc_sp.at[didx_t.at[j - 1]],
                                  sems_b).wait()
        pltpu.async_copy(h_hbm.at[sidx_b], rows_b, semg_b)
        pltpu.make_async_copy(h_hbm.at[sidx_a], rows_a, semg_a).wait()
        pltpu.async_copy(rows_a, acc_sp.at[didx_t.at[j]], sems_a,
                         add=True)

        @pl.when(jj < npair - 1)
        def _():
            pltpu.sync_copy(ei_hbm.at[0, cb + j + 2], sidx_a)
            pltpu.make_async_copy(rows_a, acc_sp.at[didx_t.at[j]],
                                  sems_a).wait()
            pltpu.async_copy(h_hbm.at[sidx_a], rows_a, semg_a)

        @pl.when(jj == npair - 1)
        def _():
            pltpu.make_async_copy(rows_a, acc_sp.at[didx_t.at[j]],
                                  sems_a).wait()
        pltpu.make_async_copy(h_hbm.at[sidx_b], rows_b, semg_b).wait()
        pltpu.async_copy(rows_b, acc_sp.at[didx_t.at[j + 1]], sems_b,
                         add=True)

    pltpu.make_async_copy(rows_b, acc_sp.at[didx_t.at[2 * npair - 1]],
                          sems_b).wait()
    plsc.subcore_barrier()

    bufs = (rows_a, rows_b)
    sems = (sems_a, sems_b)

    def _o(r0):
        return pl.multiple_of(s * RPS + r0, 8)

    for i, (r0, rn) in enumerate(_OUT_CHUNKS):
        buf, sem = bufs[i % 2], sems[i % 2]
        if i >= 2:
            p0, pn = _OUT_CHUNKS[i - 2]
            pltpu.make_async_copy(buf, out_hbm.at[c, pl.ds(_o(p0), pn)],
                                  sem).wait()
        pltpu.sync_copy(acc_sp.at[pl.ds(_o(r0), rn)], buf)
        pltpu.async_copy(buf, out_hbm.at[c, pl.ds(_o(r0), rn)], sem)
    for i in (3, 4):
        r0, rn = _OUT_CHUNKS[i]
        pltpu.make_async_copy(bufs[i % 2],
                              out_hbm.at[c, pl.ds(_o(r0), rn)],
                              sems[i % 2]).wait()


_spmm_call = pl.kernel(
    _spmm_body,
    out_type=jax.ShapeDtypeStruct((NC, NPA, H), jnp.float32),
    mesh=_mesh,
    scratch_types=[
        pltpu.VMEM((CH,), jnp.int32),
        pltpu.VMEM((CH,), jnp.int32),
        pltpu.VMEM((CPW, CH), jnp.int32),
        pltpu.VMEM((CH, H), jnp.float32),
        pltpu.VMEM((CH, H), jnp.float32),
        pltpu.VMEM_SHARED((NPA, H), jnp.float32),
        pltpu.SemaphoreType.DMA,
        pltpu.SemaphoreType.DMA,
        pltpu.SemaphoreType.DMA,
        pltpu.SemaphoreType.DMA,
        pltpu.SemaphoreType.DMA,
    ],
)


def _silu(v):
    return v / (1.0 + jnp.exp(-v))


def _dinv_of(deg_ref):
    return lax.rsqrt(deg_ref[0] + deg_ref[1] + 1.0)


def _tcA1_body(x_ref, w_ref, o_ref):
    o_ref[...] = jnp.dot(x_ref[...], w_ref[...],
                         preferred_element_type=jnp.float32)


def _tcA2_body(deg_ref, g_ref, o_ref):
    o_ref[...] = g_ref[...] * _dinv_of(deg_ref)[:, None]


def _tcB_body(deg_ref, acc_ref, hp_ref, b_ref, w_ref, o_ref):
    dinv = _dinv_of(deg_ref)
    y = (acc_ref[0] + acc_ref[1] + hp_ref[...]) * dinv[:, None] + b_ref[...]
    h1 = _silu(y)
    o_ref[...] = jnp.dot(h1, w_ref[...],
                         preferred_element_type=jnp.float32) * dinv[:, None]


def _tcC_body(deg_ref, acc_ref, hp_ref, b_ref, o_ref):
    dinv = _dinv_of(deg_ref)
    y = (acc_ref[0] + acc_ref[1] + hp_ref[...]) * dinv[:, None] + b_ref[...]
    o_ref[...] = _silu(y) * dinv[:, None]


def _tcD_body(deg_ref, acc_ref, hp_ref, bt_ref,
              muw_ref, mub_ref, lvw_ref, lvb_ref, eps_ref,
              d0w_ref, d0b_ref, d1w_ref, d1b_ref, mxw_ref, mxb_ref, lx_ref,
              omu_ref, olv_ref, omx_ref, olx_ref, qp_ref, cnt_ref):
    i = pl.program_id(0)

    @pl.when(i == 0)
    def _():
        qp_ref[...] = jnp.zeros_like(qp_ref)
        cnt_ref[...] = jnp.zeros_like(cnt_ref)

    dinv = _dinv_of(deg_ref)
    y3 = (acc_ref[0] + acc_ref[1] + hp_ref[...]) * dinv[:, None]
    gids = lax.broadcasted_iota(jnp.int32, (G, BR), 0)
    msk = (bt_ref[...] == gids).astype(jnp.float32)
    qp_ref[...] += jnp.dot(msk, y3, preferred_element_type=jnp.float32)
    cnt_ref[...] += jnp.sum(msk, axis=1)[None, :]

    @pl.when(i == NBLK - 1)
    def _():
        qp = qp_ref[...] / jnp.maximum(cnt_ref[0], 1.0)[:, None]
        mu = jnp.dot(qp, muw_ref[...],
                     preferred_element_type=jnp.float32) + mub_ref[...]
        lv = jnp.dot(qp, lvw_ref[...],
                     preferred_element_type=jnp.float32) + lvb_ref[...]
        z = mu + jnp.exp(0.5 * lv) * eps_ref[...]
        hd = jnp.tanh(jnp.dot(z, d0w_ref[...],
                              preferred_element_type=jnp.float32) + d0b_ref[...])
        hd = jnp.tanh(jnp.dot(hd, d1w_ref[...],
                              preferred_element_type=jnp.float32) + d1b_ref[...])
        mx = jnp.dot(hd, mxw_ref[...],
                     preferred_element_type=jnp.float32) + mxb_ref[...]
        omu_ref[...] = mu
        olv_ref[...] = lv
        omx_ref[...] = mx
        olx_ref[...] = jnp.broadcast_to(lx_ref[...], (G, OUT))


def _whole(shape):
    nd = len(shape)
    return pl.BlockSpec(shape, lambda i: (0,) * nd)


_deg_spec = pl.BlockSpec((2, BR), lambda i: (0, i))
_row_spec = pl.BlockSpec((BR, H), lambda i: (i, 0))
_acc_spec = pl.BlockSpec((2, BR, H), lambda i: (0, i, 0))

_tcA1_call = pl.pallas_call(
    _tcA1_body,
    grid=(NBLK,),
    in_specs=[_row_spec, _whole((D, H))],
    out_specs=_row_spec,
    out_shape=jax.ShapeDtypeStruct((NPA, H), jnp.float32),
)

_tcA2_call = pl.pallas_call(
    _tcA2_body,
    grid=(NBLK,),
    in_specs=[_deg_spec, _row_spec],
    out_specs=_row_spec,
    out_shape=jax.ShapeDtypeStruct((NPA, H), jnp.float32),
)

_tcB_call = pl.pallas_call(
    _tcB_body,
    grid=(NBLK,),
    in_specs=[_deg_spec, _acc_spec, _row_spec, _whole((1, H)), _whole((H, H))],
    out_specs=_row_spec,
    out_shape=jax.ShapeDtypeStruct((NPA, H), jnp.float32),
)

_tcC_call = pl.pallas_call(
    _tcC_body,
    grid=(NBLK,),
    in_specs=[_deg_spec, _acc_spec, _row_spec, _whole((1, H))],
    out_specs=_row_spec,
    out_shape=jax.ShapeDtypeStruct((NPA, H), jnp.float32),
)

_tcD_call = pl.pallas_call(
    _tcD_body,
    grid=(NBLK,),
    in_specs=[_deg_spec, _acc_spec, _row_spec,
              pl.BlockSpec((1, BR), lambda i: (0, i)),
              _whole((H, Z)), _whole((1, Z)), _whole((H, Z)), _whole((1, Z)),
              _whole((G, Z)),
              _whole((Z, DH)), _whole((1, DH)), _whole((DH, DH)),
              _whole((1, DH)), _whole((DH, OUT)), _whole((1, OUT)),
              _whole((1, OUT))],
    out_specs=[_whole((G, Z)), _whole((G, Z)), _whole((G, OUT)),
               _whole((G, OUT))],
    out_shape=[jax.ShapeDtypeStruct((G, Z), jnp.float32),
               jax.ShapeDtypeStruct((G, Z), jnp.float32),
               jax.ShapeDtypeStruct((G, OUT), jnp.float32),
               jax.ShapeDtypeStruct((G, OUT), jnp.float32)],
    scratch_shapes=[pltpu.VMEM((G, H), jnp.float32),
                    pltpu.VMEM((1, G), jnp.float32)],
)


def kernel(x, edge_index, batch, gc0_w, gc0_b, gc1_w, gc1_b, mu_w, mu_b,
           lv_w, lv_b, d0_w, d0_b, d1_w, d1_b, mx_w, mx_b, logvar_x_param):
    xp = jnp.pad(x, ((0, NPA - N), (0, 0)))
    bt = jnp.pad(batch, (0, NPA - N), constant_values=G).reshape(1, NPA)
    ei3 = edge_index.reshape(2, NCH, CH)
    eps = jax.random.normal(jax.random.key(42), (G, Z), jnp.float32)
    b0 = gc0_b.reshape(1, H)
    b1 = gc1_b.reshape(1, H)

    g0 = _tcA1_call(xp, gc0_w)
    degs = _deg_call(ei3)
    h0p = _tcA2_call(degs, g0)
    acc1 = _spmm_call(h0p, ei3)
    h1p = _tcB_call(degs, acc1, h0p, b0, gc1_w)
    acc2 = _spmm_call(h1p, ei3)
    h2p = _tcC_call(degs, acc2, h1p, b1)
    acc3 = _spmm_call(h2p, ei3)
    mu_zp, logvar_zp, mu_x, logvar_x = _tcD_call(
        degs, acc3, h2p, bt,
        mu_w, mu_b.reshape(1, Z), lv_w, lv_b.reshape(1, Z), eps,
        d0_w, d0_b.reshape(1, DH), d1_w, d1_b.reshape(1, DH),
        mx_w, mx_b.reshape(1, OUT), logvar_x_param.reshape(1, OUT))
    return (mu_zp, logvar_zp, mu_x, logvar_x)


# speedup vs baseline: 1.0694x; 1.0694x over previous
import jax
import jax.numpy as jnp
from jax import lax
from jax.experimental import pallas as pl
from jax.experimental.pallas import tpu as pltpu
from jax.experimental.pallas import tpu_sc as plsc

N = 10000
E = 320000
D = 128
H = 128
Z = 64
DH = 256
OUT = 231
G = 64

NC = 2
NS = 16
NW = NC * NS
CH = 128
NCH = E // CH
CPW = 80
NPA = 10240
RPS = NPA // NS
NPD = 10240
BR = 1024
NBLK = NPA // BR

_mesh = plsc.VectorSubcoreMesh(core_axis_name="c", subcore_axis_name="s")


def _zero_rows(buf, nrows):
    z = jnp.zeros((16,), jnp.float32)

    @pl.loop(0, nrows)
    def _(r):
        for k in range(8):
            buf[r, pl.ds(k * 16, 16)] = z


_OUT_CHUNKS = [(0, 128), (128, 128), (256, 128), (384, 128), (512, 128)]


def _stage_dst_table(ei_hbm, didx_t, wid, semi, wait):
    last = NW - 1

    @pl.when(wid < last)
    def _():
        off = pl.multiple_of(wid * CPW, 8)
        if wait:
            pltpu.make_async_copy(ei_hbm.at[1, pl.ds(off, CPW)], didx_t,
                                  semi).wait()
        else:
            pltpu.async_copy(ei_hbm.at[1, pl.ds(off, CPW)], didx_t, semi)

    @pl.when(wid == last)
    def _():
        ops = [(ei_hbm.at[1, pl.ds(NCH - 20, 16)], didx_t.at[pl.ds(0, 16)])]
        ops += [(ei_hbm.at[1, NCH - 4 + k], didx_t.at[16 + k])
                for k in range(4)]
        for src, dst in ops:
            if wait:
                pltpu.make_async_copy(src, dst, semi).wait()
            else:
                pltpu.async_copy(src, dst, semi)


def _deg_body(ei_hbm, deg_out, didx_t, ones_v, zrow_v, deg_sp, semi):
    c = lax.axis_index("c")
    s = lax.axis_index("s")
    wid = s * NC + c
    cb = jnp.minimum(wid * CPW, NCH - 20)
    nch = jnp.where(wid == NW - 1, 20, CPW)

    _stage_dst_table(ei_hbm, didx_t, wid, semi, wait=False)

    @pl.loop(0, NPD // NS // 16)
    def _(r):
        zrow_v[pl.ds(r * 16, 16)] = jnp.zeros((16,), jnp.float32)
    for k in range(8):
        ones_v[pl.ds(k * 16, 16)] = jnp.ones((16,), jnp.float32)
    pltpu.sync_copy(zrow_v,
                    deg_sp.at[pl.ds(pl.multiple_of(s * (NPD // NS), 8),
                                    NPD // NS)])
    _stage_dst_table(ei_hbm, didx_t, wid, semi, wait=True)
    plsc.subcore_barrier()

    @pl.loop(0, nch)
    def _(j):
        pltpu.sync_copy(ones_v, deg_sp.at[didx_t.at[j]], add=True)

    plsc.subcore_barrier()
    pltpu.sync_copy(
        deg_sp.at[pl.ds(pl.multiple_of(s * (NPD // NS), 8), NPD // NS)],
        zrow_v)
    pltpu.sync_copy(
        zrow_v,
        deg_out.at[c, pl.ds(pl.multiple_of(s * (NPD // NS), 8), NPD // NS)])


_deg_call = pl.kernel(
    _deg_body,
    out_type=jax.ShapeDtypeStruct((NC, NPD), jnp.float32),
    mesh=_mesh,
    scratch_types=[
        pltpu.VMEM((CPW, CH), jnp.int32),
        pltpu.VMEM((CH,), jnp.float32),
        pltpu.VMEM((NPD // NS,), jnp.float32),
        pltpu.VMEM_SHARED((NPD,), jnp.float32),
        pltpu.SemaphoreType.DMA,
    ],
)


def _spmm_body(h_hbm, ei_hbm, out_hbm,
               sidx_a, sidx_b, didx_t, rows_a, rows_b, acc_sp,
               semg_a, semg_b, sems_a, sems_b, semi):
    c = lax.axis_index("c")
    s = lax.axis_index("s")
    wid = s * NC + c
    cb = jnp.minimum(wid * CPW, NCH - 20)
    npair = jnp.where(wid == NW - 1, 10, CPW // 2)

    _stage_dst_table(ei_hbm, didx_t, wid, semi, wait=False)
    pltpu.sync_copy(ei_hbm.at[0, cb], sidx_a)
    pltpu.async_copy(h_hbm.at[sidx_a], rows_a, semg_a)

    _zero_rows(rows_b, CH)
    for (r0, rn) in _OUT_CHUNKS:
        off = pl.multiple_of(s * RPS + r0, 8)
        pltpu.async_copy(rows_b, acc_sp.at[pl.ds(off, rn)], sems_b)
    for (r0, rn) in _OUT_CHUNKS:
        off = pl.multiple_of(s * RPS + r0, 8)
        pltpu.make_async_copy(rows_b, acc_sp.at[pl.ds(off, rn)],
                              sems_b).wait()
    _stage_dst_table(ei_hbm, didx_t, wid, semi, wait=True)
    plsc.subcore_barrier()

    @pl.loop(0, npair)
    def _(jj):
        j = jj * 2
        pltpu.sync_copy(ei_hbm.at[0, cb + j + 1], sidx_b)

        @pl.when(jj > 0)
        def _():
            pltpu.make_async_copy(rows_b, acc_sp.at[didx_t.at[j - 1]],
                                  sems_b).wait()
        pltpu.async_copy(h_hbm.at[sidx_b], rows_b, semg_b)
        pltpu.make_async_copy(h_hbm.at[sidx_a], rows_a, semg_a).wait()
        pltpu.async_copy(rows_a, acc_sp.at[didx_t.at[j]], sems_a,
                         add=True)

        @pl.when(jj < npair - 1)
        def _():
            pltpu.sync_copy(ei_hbm.at[0, cb + j + 2], sidx_a)
            pltpu.make_async_copy(rows_a, acc_sp.at[didx_t.at[j]],
                                  sems_a).wait()
            pltpu.async_copy(h_hbm.at[sidx_a], rows_a, semg_a)

        @pl.when(jj == npair - 1)
        def _():
            pltpu.make_async_copy(rows_a, acc_sp.at[didx_t.at[j]],
                                  sems_a).wait()
        pltpu.make_async_copy(h_hbm.at[sidx_b], rows_b, semg_b).wait()
        pltpu.async_copy(rows_b, acc_sp.at[didx_t.at[j + 1]], sems_b,
                         add=True)

    pltpu.make_async_copy(rows_b, acc_sp.at[didx_t.at[2 * npair - 1]],
                          sems_b).wait()
    plsc.subcore_barrier()

    bufs = (rows_a, rows_b)
    sems = (sems_a, sems_b)

    def _o(r0):
        return pl.multiple_of(s * RPS + r0, 8)

    for i, (r0, rn) in enumerate(_OUT_CHUNKS):
        buf, sem = bufs[i % 2], sems[i % 2]
        if i >= 2:
            p0, pn = _OUT_CHUNKS[i - 2]
            pltpu.make_async_copy(buf, out_hbm.at[c, pl.ds(_o(p0), pn)],
                                  sem).wait()
        pltpu.sync_copy(acc_sp.at[pl.ds(_o(r0), rn)], buf)
        pltpu.async_copy(buf, out_hbm.at[c, pl.ds(_o(r0), rn)], sem)
    for i in (3, 4):
        r0, rn = _OUT_CHUNKS[i]
        pltpu.make_async_copy(bufs[i % 2],
                              out_hbm.at[c, pl.ds(_o(r0), rn)],
                              sems[i % 2]).wait()


_spmm_call = pl.kernel(
    _spmm_body,
    out_type=jax.ShapeDtypeStruct((NC, NPA, H), jnp.float32),
    mesh=_mesh,
    scratch_types=[
        pltpu.VMEM((CH,), jnp.int32),
        pltpu.VMEM((CH,), jnp.int32),
        pltpu.VMEM((CPW, CH), jnp.int32),
        pltpu.VMEM((CH, H), jnp.float32),
        pltpu.VMEM((CH, H), jnp.float32),
        pltpu.VMEM_SHARED((NPA, H), jnp.float32),
        pltpu.SemaphoreType.DMA,
        pltpu.SemaphoreType.DMA,
        pltpu.SemaphoreType.DMA,
        pltpu.SemaphoreType.DMA,
        pltpu.SemaphoreType.DMA,
    ],
)


def _silu(v):
    return v / (1.0 + jnp.exp(-v))


def _dinv_of(deg_ref):
    return lax.rsqrt(deg_ref[0] + deg_ref[1] + 1.0)


def _tcA_body(deg_ref, x_ref, w_ref, o_ref):
    dinv = _dinv_of(deg_ref)
    h = jnp.dot(x_ref[...], w_ref[...], preferred_element_type=jnp.float32)
    o_ref[...] = h * dinv[:, None]


def _tcB_body(deg_ref, acc_ref, hp_ref, b_ref, w_ref, o_ref):
    dinv = _dinv_of(deg_ref)
    y = (acc_ref[0] + acc_ref[1] + hp_ref[...]) * dinv[:, None] + b_ref[...]
    h1 = _silu(y)
    o_ref[...] = jnp.dot(h1, w_ref[...],
                         preferred_element_type=jnp.float32) * dinv[:, None]


def _tcC_body(deg_ref, acc_ref, hp_ref, b_ref, o_ref):
    dinv = _dinv_of(deg_ref)
    y = (acc_ref[0] + acc_ref[1] + hp_ref[...]) * dinv[:, None] + b_ref[...]
    o_ref[...] = _silu(y) * dinv[:, None]


def _tcD_body(deg_ref, acc_ref, hp_ref, bt_ref,
              muw_ref, mub_ref, lvw_ref, lvb_ref, eps_ref,
              d0w_ref, d0b_ref, d1w_ref, d1b_ref, mxw_ref, mxb_ref, lx_ref,
              omu_ref, olv_ref, omx_ref, olx_ref, qp_ref, cnt_ref):
    i = pl.program_id(0)

    @pl.when(i == 0)
    def _():
        qp_ref[...] = jnp.zeros_like(qp_ref)
        cnt_ref[...] = jnp.zeros_like(cnt_ref)

    dinv = _dinv_of(deg_ref)
    y3 = (acc_ref[0] + acc_ref[1] + hp_ref[...]) * dinv[:, None]
    gids = lax.broadcasted_iota(jnp.int32, (G, BR), 0)
    msk = (bt_ref[...] == gids).astype(jnp.float32)
    qp_ref[...] += jnp.dot(msk, y3, preferred_element_type=jnp.float32)
    cnt_ref[...] += jnp.sum(msk, axis=1)[None, :]

    @pl.when(i == NBLK - 1)
    def _():
        qp = qp_ref[...] / jnp.maximum(cnt_ref[0], 1.0)[:, None]
        mu = jnp.dot(qp, muw_ref[...],
                     preferred_element_type=jnp.float32) + mub_ref[...]
        lv = jnp.dot(qp, lvw_ref[...],
                     preferred_element_type=jnp.float32) + lvb_ref[...]
        z = mu + jnp.exp(0.5 * lv) * eps_ref[...]
        hd = jnp.tanh(jnp.dot(z, d0w_ref[...],
                              preferred_element_type=jnp.float32) + d0b_ref[...])
        hd = jnp.tanh(jnp.dot(hd, d1w_ref[...],
                              preferred_element_type=jnp.float32) + d1b_ref[...])
        mx = jnp.dot(hd, mxw_ref[...],
                     preferred_element_type=jnp.float32) + mxb_ref[...]
        omu_ref[...] = mu
        olv_ref[...] = lv
        omx_ref[...] = mx
        olx_ref[...] = jnp.broadcast_to(lx_ref[...], (G, OUT))


def _whole(shape):
    nd = len(shape)
    return pl.BlockSpec(shape, lambda i: (0,) * nd)


_deg_spec = pl.BlockSpec((2, BR), lambda i: (0, i))
_row_spec = pl.BlockSpec((BR, H), lambda i: (i, 0))
_acc_spec = pl.BlockSpec((2, BR, H), lambda i: (0, i, 0))

_tcA_call = pl.pallas_call(
    _tcA_body,
    grid=(NBLK,),
    in_specs=[_deg_spec, _row_spec, _whole((D, H))],
    out_specs=_row_spec,
    out_shape=jax.ShapeDtypeStruct((NPA, H), jnp.float32),
)

_tcB_call = pl.pallas_call(
    _tcB_body,
    grid=(NBLK,),
    in_specs=[_deg_spec, _acc_spec, _row_spec, _whole((1, H)), _whole((H, H))],
    out_specs=_row_spec,
    out_shape=jax.ShapeDtypeStruct((NPA, H), jnp.float32),
)

_tcC_call = pl.pallas_call(
    _tcC_body,
    grid=(NBLK,),
    in_specs=[_deg_spec, _acc_spec, _row_spec, _whole((1, H))],
    out_specs=_row_spec,
    out_shape=jax.ShapeDtypeStruct((NPA, H), jnp.float32),
)

_tcD_call = pl.pallas_call(
    _tcD_body,
    grid=(NBLK,),
    in_specs=[_deg_spec, _acc_spec, _row_spec,
              pl.BlockSpec((1, BR), lambda i: (0, i)),
              _whole((H, Z)), _whole((1, Z)), _whole((H, Z)), _whole((1, Z)),
              _whole((G, Z)),
              _whole((Z, DH)), _whole((1, DH)), _whole((DH, DH)),
              _whole((1, DH)), _whole((DH, OUT)), _whole((1, OUT)),
              _whole((1, OUT))],
    out_specs=[_whole((G, Z)), _whole((G, Z)), _whole((G, OUT)),
               _whole((G, OUT))],
    out_shape=[jax.ShapeDtypeStruct((G, Z), jnp.float32),
               jax.ShapeDtypeStruct((G, Z), jnp.float32),
               jax.ShapeDtypeStruct((G, OUT), jnp.float32),
               jax.ShapeDtypeStruct((G, OUT), jnp.float32)],
    scratch_shapes=[pltpu.VMEM((G, H), jnp.float32),
                    pltpu.VMEM((1, G), jnp.float32)],
)


def kernel(x, edge_index, batch, gc0_w, gc0_b, gc1_w, gc1_b, mu_w, mu_b,
           lv_w, lv_b, d0_w, d0_b, d1_w, d1_b, mx_w, mx_b, logvar_x_param):
    xp = jnp.pad(x, ((0, NPA - N), (0, 0)))
    bt = jnp.pad(batch, (0, NPA - N), constant_values=G).reshape(1, NPA)
    ei3 = edge_index.reshape(2, NCH, CH)
    eps = jax.random.normal(jax.random.key(42), (G, Z), jnp.float32)
    b0 = gc0_b.reshape(1, H)
    b1 = gc1_b.reshape(1, H)

    degs = _deg_call(ei3)
    h0p = _tcA_call(degs, xp, gc0_w)
    acc1 = _spmm_call(h0p, ei3)
    h1p = _tcB_call(degs, acc1, h0p, b0, gc1_w)
    acc2 = _spmm_call(h1p, ei3)
    h2p = _tcC_call(degs, acc2, h1p, b1)
    acc3 = _spmm_call(h2p, ei3)
    mu_zp, logvar_zp, mu_x, logvar_x = _tcD_call(
        degs, acc3, h2p, bt,
        mu_w, mu_b.reshape(1, Z), lv_w, lv_b.reshape(1, Z), eps,
        d0_w, d0_b.reshape(1, DH), d1_w, d1_b.reshape(1, DH),
        mx_w, mx_b.reshape(1, OUT), logvar_x_param.reshape(1, OUT))
    return (mu_zp, logvar_zp, mu_x, logvar_x)


# speedup vs baseline: 1.0947x; 1.0237x over previous
import jax
import jax.numpy as jnp
from jax import lax
from jax.experimental import pallas as pl
from jax.experimental.pallas import tpu as pltpu
from jax.experimental.pallas import tpu_sc as plsc

N = 10000
E = 320000
D = 128
H = 128
Z = 64
DH = 256
OUT = 231
G = 64

NC = 2
NS = 16
NW = NC * NS
CH = 128
NCH = E // CH
CPW = 80
NPA = 10240
RPS = NPA // NS
NPD = 10240
BR = 2048
NBLK = NPA // BR

_mesh = plsc.VectorSubcoreMesh(core_axis_name="c", subcore_axis_name="s")


def _zero_rows(buf, nrows):
    z = jnp.zeros((16,), jnp.float32)

    @pl.loop(0, nrows)
    def _(r):
        for k in range(8):
            buf[r, pl.ds(k * 16, 16)] = z


_OUT_CHUNKS = [(0, 128), (128, 128), (256, 128), (384, 128), (512, 128)]


def _stage_dst_table(ei_hbm, didx_t, wid, semi, wait):
    last = NW - 1

    @pl.when(wid < last)
    def _():
        off = pl.multiple_of(wid * CPW, 8)
        if wait:
            pltpu.make_async_copy(ei_hbm.at[1, pl.ds(off, CPW)], didx_t,
                                  semi).wait()
        else:
            pltpu.async_copy(ei_hbm.at[1, pl.ds(off, CPW)], didx_t, semi)

    @pl.when(wid == last)
    def _():
        ops = [(ei_hbm.at[1, pl.ds(NCH - 20, 16)], didx_t.at[pl.ds(0, 16)])]
        ops += [(ei_hbm.at[1, NCH - 4 + k], didx_t.at[16 + k])
                for k in range(4)]
        for src, dst in ops:
            if wait:
                pltpu.make_async_copy(src, dst, semi).wait()
            else:
                pltpu.async_copy(src, dst, semi)


def _deg_body(ei_hbm, deg_out, didx_t, ones_v, zrow_v, deg_sp, semi):
    c = lax.axis_index("c")
    s = lax.axis_index("s")
    wid = s * NC + c
    cb = jnp.minimum(wid * CPW, NCH - 20)
    nch = jnp.where(wid == NW - 1, 20, CPW)

    _stage_dst_table(ei_hbm, didx_t, wid, semi, wait=False)

    @pl.loop(0, NPD // NS // 16)
    def _(r):
        zrow_v[pl.ds(r * 16, 16)] = jnp.zeros((16,), jnp.float32)
    for k in range(8):
        ones_v[pl.ds(k * 16, 16)] = jnp.ones((16,), jnp.float32)
    pltpu.sync_copy(zrow_v,
                    deg_sp.at[pl.ds(pl.multiple_of(s * (NPD // NS), 8),
                                    NPD // NS)])
    _stage_dst_table(ei_hbm, didx_t, wid, semi, wait=True)
    plsc.subcore_barrier()

    @pl.loop(0, nch)
    def _(j):
        pltpu.sync_copy(ones_v, deg_sp.at[didx_t.at[j]], add=True)

    plsc.subcore_barrier()
    pltpu.sync_copy(
        deg_sp.at[pl.ds(pl.multiple_of(s * (NPD // NS), 8), NPD // NS)],
        zrow_v)
    pltpu.sync_copy(
        zrow_v,
        deg_out.at[c, pl.ds(pl.multiple_of(s * (NPD // NS), 8), NPD // NS)])


_deg_call = pl.kernel(
    _deg_body,
    out_type=jax.ShapeDtypeStruct((NC, NPD), jnp.float32),
    mesh=_mesh,
    scratch_types=[
        pltpu.VMEM((CPW, CH), jnp.int32),
        pltpu.VMEM((CH,), jnp.float32),
        pltpu.VMEM((NPD // NS,), jnp.float32),
        pltpu.VMEM_SHARED((NPD,), jnp.float32),
        pltpu.SemaphoreType.DMA,
    ],
)


def _spmm_body(h_hbm, ei_hbm, out_hbm,
               sidx_a, sidx_b, didx_t, rows_a, rows_b, acc_sp,
               semg_a, semg_b, sems_a, sems_b, semi):
    c = lax.axis_index("c")
    s = lax.axis_index("s")
    wid = s * NC + c
    cb = jnp.minimum(wid * CPW, NCH - 20)
    npair = jnp.where(wid == NW - 1, 10, CPW // 2)

    _stage_dst_table(ei_hbm, didx_t, wid, semi, wait=False)
    pltpu.sync_copy(ei_hbm.at[0, cb], sidx_a)
    pltpu.async_copy(h_hbm.at[sidx_a], rows_a, semg_a)

    _zero_rows(rows_b, CH)
    for (r0, rn) in _OUT_CHUNKS:
        off = pl.multiple_of(s * RPS + r0, 8)
        pltpu.async_copy(rows_b, acc_sp.at[pl.ds(off, rn)], sems_b)
    for (r0, rn) in _OUT_CHUNKS:
        off = pl.multiple_of(s * RPS + r0, 8)
        pltpu.make_async_copy(rows_b, acc_sp.at[pl.ds(off, rn)],
                              sems_b).wait()
    _stage_dst_table(ei_hbm, didx_t, wid, semi, wait=True)
    plsc.subcore_barrier()

    @pl.loop(0, npair)
    def _(jj):
        j = jj * 2
        pltpu.sync_copy(ei_hbm.at[0, cb + j + 1], sidx_b)

        @pl.when(jj > 0)
        def _():
            pltpu.make_async_copy(rows_b, acc_sp.at[didx_t.at[j - 1]],
                                  sems_b).wait()
        pltpu.async_copy(h_hbm.at[sidx_b], rows_b, semg_b)
        pltpu.make_async_copy(h_hbm.at[sidx_a], rows_a, semg_a).wait()
        pltpu.async_copy(rows_a, acc_sp.at[didx_t.at[j]], sems_a,
                         add=True)

        @pl.when(jj < npair - 1)
        def _():
            pltpu.sync_copy(ei_hbm.at[0, cb + j + 2], sidx_a)
            pltpu.make_async_copy(rows_a, acc_sp.at[didx_t.at[j]],
                                  sems_a).wait()
            pltpu.async_copy(h_hbm.at[sidx_a], rows_a, semg_a)

        @pl.when(jj == npair - 1)
        def _():
            pltpu.make_async_copy(rows_a, acc_sp.at[didx_t.at[j]],
                                  sems_a).wait()
        pltpu.make_async_copy(h_hbm.at[sidx_b], rows_b, semg_b).wait()
        pltpu.async_copy(rows_b, acc_sp.at[didx_t.at[j + 1]], sems_b,
                         add=True)

    pltpu.make_async_copy(rows_b, acc_sp.at[didx_t.at[2 * npair - 1]],
                          sems_b).wait()
    plsc.subcore_barrier()

    bufs = (rows_a, rows_b)
    sems = (sems_a, sems_b)

    def _o(r0):
        return pl.multiple_of(s * RPS + r0, 8)

    for i, (r0, rn) in enumerate(_OUT_CHUNKS):
        buf, sem = bufs[i % 2], sems[i % 2]
        if i >= 2:
            p0, pn = _OUT_CHUNKS[i - 2]
            pltpu.make_async_copy(buf, out_hbm.at[c, pl.ds(_o(p0), pn)],
                                  sem).wait()
        pltpu.sync_copy(acc_sp.at[pl.ds(_o(r0), rn)], buf)
        pltpu.async_copy(buf, out_hbm.at[c, pl.ds(_o(r0), rn)], sem)
    for i in (3, 4):
        r0, rn = _OUT_CHUNKS[i]
        pltpu.make_async_copy(bufs[i % 2],
                              out_hbm.at[c, pl.ds(_o(r0), rn)],
                              sems[i % 2]).wait()


_spmm_call = pl.kernel(
    _spmm_body,
    out_type=jax.ShapeDtypeStruct((NC, NPA, H), jnp.float32),
    mesh=_mesh,
    scratch_types=[
        pltpu.VMEM((CH,), jnp.int32),
        pltpu.VMEM((CH,), jnp.int32),
        pltpu.VMEM((CPW, CH), jnp.int32),
        pltpu.VMEM((CH, H), jnp.float32),
        pltpu.VMEM((CH, H), jnp.float32),
        pltpu.VMEM_SHARED((NPA, H), jnp.float32),
        pltpu.SemaphoreType.DMA,
        pltpu.SemaphoreType.DMA,
        pltpu.SemaphoreType.DMA,
        pltpu.SemaphoreType.DMA,
        pltpu.SemaphoreType.DMA,
    ],
)


def _silu(v):
    return v / (1.0 + jnp.exp(-v))


def _dinv_of(deg_ref):
    return lax.rsqrt(deg_ref[0] + deg_ref[1] + 1.0)


def _tcA_body(deg_ref, x_ref, w_ref, o_ref):
    dinv = _dinv_of(deg_ref)
    h = jnp.dot(x_ref[...], w_ref[...], preferred_element_type=jnp.float32)
    o_ref[...] = h * dinv[:, None]


def _tcB_body(deg_ref, acc_ref, hp_ref, b_ref, w_ref, o_ref):
    dinv = _dinv_of(deg_ref)
    y = (acc_ref[0] + acc_ref[1] + hp_ref[...]) * dinv[:, None] + b_ref[...]
    h1 = _silu(y)
    o_ref[...] = jnp.dot(h1, w_ref[...],
                         preferred_element_type=jnp.float32) * dinv[:, None]


def _tcC_body(deg_ref, acc_ref, hp_ref, b_ref, o_ref):
    dinv = _dinv_of(deg_ref)
    y = (acc_ref[0] + acc_ref[1] + hp_ref[...]) * dinv[:, None] + b_ref[...]
    o_ref[...] = _silu(y) * dinv[:, None]


def _tcD_body(deg_ref, acc_ref, hp_ref, bt_ref,
              muw_ref, mub_ref, lvw_ref, lvb_ref, eps_ref,
              d0w_ref, d0b_ref, d1w_ref, d1b_ref, mxw_ref, mxb_ref, lx_ref,
              omu_ref, olv_ref, omx_ref, olx_ref, qp_ref, cnt_ref):
    i = pl.program_id(0)

    @pl.when(i == 0)
    def _():
        qp_ref[...] = jnp.zeros_like(qp_ref)
        cnt_ref[...] = jnp.zeros_like(cnt_ref)

    dinv = _dinv_of(deg_ref)
    y3 = (acc_ref[0] + acc_ref[1] + hp_ref[...]) * dinv[:, None]
    gids = lax.broadcasted_iota(jnp.int32, (G, BR), 0)
    msk = (bt_ref[...] == gids).astype(jnp.float32)
    qp_ref[...] += jnp.dot(msk, y3, preferred_element_type=jnp.float32)
    cnt_ref[...] += jnp.sum(msk, axis=1)[None, :]

    @pl.when(i == NBLK - 1)
    def _():
        qp = qp_ref[...] / jnp.maximum(cnt_ref[0], 1.0)[:, None]
        mu = jnp.dot(qp, muw_ref[...],
                     preferred_element_type=jnp.float32) + mub_ref[...]
        lv = jnp.dot(qp, lvw_ref[...],
                     preferred_element_type=jnp.float32) + lvb_ref[...]
        z = mu + jnp.exp(0.5 * lv) * eps_ref[...]
        hd = jnp.tanh(jnp.dot(z, d0w_ref[...],
                              preferred_element_type=jnp.float32) + d0b_ref[...])
        hd = jnp.tanh(jnp.dot(hd, d1w_ref[...],
                              preferred_element_type=jnp.float32) + d1b_ref[...])
        mx = jnp.dot(hd, mxw_ref[...],
                     preferred_element_type=jnp.float32) + mxb_ref[...]
        omu_ref[...] = mu
        olv_ref[...] = lv
        omx_ref[...] = mx
        olx_ref[...] = jnp.broadcast_to(lx_ref[...], (G, OUT))


def _whole(shape):
    nd = len(shape)
    return pl.BlockSpec(shape, lambda i: (0,) * nd)


_deg_spec = pl.BlockSpec((2, BR), lambda i: (0, i))
_row_spec = pl.BlockSpec((BR, H), lambda i: (i, 0))
_acc_spec = pl.BlockSpec((2, BR, H), lambda i: (0, i, 0))

_tcA_call = pl.pallas_call(
    _tcA_body,
    grid=(NBLK,),
    in_specs=[_deg_spec, _row_spec, _whole((D, H))],
    out_specs=_row_spec,
    out_shape=jax.ShapeDtypeStruct((NPA, H), jnp.float32),
)

_tcB_call = pl.pallas_call(
    _tcB_body,
    grid=(NBLK,),
    in_specs=[_deg_spec, _acc_spec, _row_spec, _whole((1, H)), _whole((H, H))],
    out_specs=_row_spec,
    out_shape=jax.ShapeDtypeStruct((NPA, H), jnp.float32),
)

_tcC_call = pl.pallas_call(
    _tcC_body,
    grid=(NBLK,),
    in_specs=[_deg_spec, _acc_spec, _row_spec, _whole((1, H))],
    out_specs=_row_spec,
    out_shape=jax.ShapeDtypeStruct((NPA, H), jnp.float32),
)

_tcD_call = pl.pallas_call(
    _tcD_body,
    grid=(NBLK,),
    in_specs=[_deg_spec, _acc_spec, _row_spec,
              pl.BlockSpec((1, BR), lambda i: (0, i)),
              _whole((H, Z)), _whole((1, Z)), _whole((H, Z)), _whole((1, Z)),
              _whole((G, Z)),
              _whole((Z, DH)), _whole((1, DH)), _whole((DH, DH)),
              _whole((1, DH)), _whole((DH, OUT)), _whole((1, OUT)),
              _whole((1, OUT))],
    out_specs=[_whole((G, Z)), _whole((G, Z)), _whole((G, OUT)),
               _whole((G, OUT))],
    out_shape=[jax.ShapeDtypeStruct((G, Z), jnp.float32),
               jax.ShapeDtypeStruct((G, Z), jnp.float32),
               jax.ShapeDtypeStruct((G, OUT), jnp.float32),
               jax.ShapeDtypeStruct((G, OUT), jnp.float32)],
    scratch_shapes=[pltpu.VMEM((G, H), jnp.float32),
                    pltpu.VMEM((1, G), jnp.float32)],
)


def kernel(x, edge_index, batch, gc0_w, gc0_b, gc1_w, gc1_b, mu_w, mu_b,
           lv_w, lv_b, d0_w, d0_b, d1_w, d1_b, mx_w, mx_b, logvar_x_param):
    xp = jnp.pad(x, ((0, NPA - N), (0, 0)))
    bt = jnp.pad(batch, (0, NPA - N), constant_values=G).reshape(1, NPA)
    ei3 = edge_index.reshape(2, NCH, CH)
    eps = jax.random.normal(jax.random.key(42), (G, Z), jnp.float32)
    b0 = gc0_b.reshape(1, H)
    b1 = gc1_b.reshape(1, H)

    degs = _deg_call(ei3)
    h0p = _tcA_call(degs, xp, gc0_w)
    acc1 = _spmm_call(h0p, ei3)
    h1p = _tcB_call(degs, acc1, h0p, b0, gc1_w)
    acc2 = _spmm_call(h1p, ei3)
    h2p = _tcC_call(degs, acc2, h1p, b1)
    acc3 = _spmm_call(h2p, ei3)
    mu_zp, logvar_zp, mu_x, logvar_x = _tcD_call(
        degs, acc3, h2p, bt,
        mu_w, mu_b.reshape(1, Z), lv_w, lv_b.reshape(1, Z), eps,
        d0_w, d0_b.reshape(1, DH), d1_w, d1_b.reshape(1, DH),
        mx_w, mx_b.reshape(1, OUT), logvar_x_param.reshape(1, OUT))
    return (mu_zp, logvar_zp, mu_x, logvar_x)


# speedup vs baseline: 1.1087x; 1.0127x over previous
import jax
import jax.numpy as jnp
from jax import lax
from jax.experimental import pallas as pl
from jax.experimental.pallas import tpu as pltpu
from jax.experimental.pallas import tpu_sc as plsc

N = 10000
E = 320000
D = 128
H = 128
Z = 64
DH = 256
OUT = 231
G = 64

NC = 2
NS = 16
NW = NC * NS
CH = 128
NCH = E // CH
CPW = 80
NPA = 10240
RPS = NPA // NS
NPD = 10240
BR = 2560
NBLK = NPA // BR

_mesh = plsc.VectorSubcoreMesh(core_axis_name="c", subcore_axis_name="s")


def _zero_rows(buf, nrows):
    z = jnp.zeros((16,), jnp.float32)

    @pl.loop(0, nrows)
    def _(r):
        for k in range(8):
            buf[r, pl.ds(k * 16, 16)] = z


_OUT_CHUNKS = [(0, 128), (128, 128), (256, 128), (384, 128), (512, 128)]


def _stage_dst_table(ei_hbm, didx_t, wid, semi, wait):
    last = NW - 1

    @pl.when(wid < last)
    def _():
        off = pl.multiple_of(wid * CPW, 8)
        if wait:
            pltpu.make_async_copy(ei_hbm.at[1, pl.ds(off, CPW)], didx_t,
                                  semi).wait()
        else:
            pltpu.async_copy(ei_hbm.at[1, pl.ds(off, CPW)], didx_t, semi)

    @pl.when(wid == last)
    def _():
        ops = [(ei_hbm.at[1, pl.ds(NCH - 20, 16)], didx_t.at[pl.ds(0, 16)])]
        ops += [(ei_hbm.at[1, NCH - 4 + k], didx_t.at[16 + k])
                for k in range(4)]
        for src, dst in ops:
            if wait:
                pltpu.make_async_copy(src, dst, semi).wait()
            else:
                pltpu.async_copy(src, dst, semi)


def _deg_body(ei_hbm, deg_out, didx_t, ones_v, zrow_v, deg_sp, semi):
    c = lax.axis_index("c")
    s = lax.axis_index("s")
    wid = s * NC + c
    cb = jnp.minimum(wid * CPW, NCH - 20)
    nch = jnp.where(wid == NW - 1, 20, CPW)

    _stage_dst_table(ei_hbm, didx_t, wid, semi, wait=False)

    @pl.loop(0, NPD // NS // 16)
    def _(r):
        zrow_v[pl.ds(r * 16, 16)] = jnp.zeros((16,), jnp.float32)
    for k in range(8):
        ones_v[pl.ds(k * 16, 16)] = jnp.ones((16,), jnp.float32)
    pltpu.sync_copy(zrow_v,
                    deg_sp.at[pl.ds(pl.multiple_of(s * (NPD // NS), 8),
                                    NPD // NS)])
    _stage_dst_table(ei_hbm, didx_t, wid, semi, wait=True)
    plsc.subcore_barrier()

    @pl.loop(0, nch)
    def _(j):
        pltpu.sync_copy(ones_v, deg_sp.at[didx_t.at[j]], add=True)

    plsc.subcore_barrier()
    pltpu.sync_copy(
        deg_sp.at[pl.ds(pl.multiple_of(s * (NPD // NS), 8), NPD // NS)],
        zrow_v)
    pltpu.sync_copy(
        zrow_v,
        deg_out.at[c, pl.ds(pl.multiple_of(s * (NPD // NS), 8), NPD // NS)])


_deg_call = pl.kernel(
    _deg_body,
    out_type=jax.ShapeDtypeStruct((NC, NPD), jnp.float32),
    mesh=_mesh,
    scratch_types=[
        pltpu.VMEM((CPW, CH), jnp.int32),
        pltpu.VMEM((CH,), jnp.float32),
        pltpu.VMEM((NPD // NS,), jnp.float32),
        pltpu.VMEM_SHARED((NPD,), jnp.float32),
        pltpu.SemaphoreType.DMA,
    ],
)


def _spmm_body(h_hbm, ei_hbm, out_hbm,
               sidx_a, sidx_b, didx_t, rows_a, rows_b, acc_sp,
               semg_a, semg_b, sems_a, sems_b, semi):
    c = lax.axis_index("c")
    s = lax.axis_index("s")
    wid = s * NC + c
    cb = jnp.minimum(wid * CPW, NCH - 20)
    npair = jnp.where(wid == NW - 1, 10, CPW // 2)

    _stage_dst_table(ei_hbm, didx_t, wid, semi, wait=False)
    pltpu.sync_copy(ei_hbm.at[0, cb], sidx_a)
    pltpu.async_copy(h_hbm.at[sidx_a], rows_a, semg_a)

    _zero_rows(rows_b, CH)
    for (r0, rn) in _OUT_CHUNKS:
        off = pl.multiple_of(s * RPS + r0, 8)
        pltpu.async_copy(rows_b, acc_sp.at[pl.ds(off, rn)], sems_b)
    for (r0, rn) in _OUT_CHUNKS:
        off = pl.multiple_of(s * RPS + r0, 8)
        pltpu.make_async_copy(rows_b, acc_sp.at[pl.ds(off, rn)],
                              sems_b).wait()
    _stage_dst_table(ei_hbm, didx_t, wid, semi, wait=True)
    plsc.subcore_barrier()

    @pl.loop(0, npair)
    def _(jj):
        j = jj * 2
        pltpu.sync_copy(ei_hbm.at[0, cb + j + 1], sidx_b)

        @pl.when(jj > 0)
        def _():
            pltpu.make_async_copy(rows_b, acc_sp.at[didx_t.at[j - 1]],
                                  sems_b).wait()
        pltpu.async_copy(h_hbm.at[sidx_b], rows_b, semg_b)
        pltpu.make_async_copy(h_hbm.at[sidx_a], rows_a, semg_a).wait()
        pltpu.async_copy(rows_a, acc_sp.at[didx_t.at[j]], sems_a,
                         add=True)

        @pl.when(jj < npair - 1)
        def _():
            pltpu.sync_copy(ei_hbm.at[0, cb + j + 2], sidx_a)
            pltpu.make_async_copy(rows_a, acc_sp.at[didx_t.at[j]],
                                  sems_a).wait()
            pltpu.async_copy(h_hbm.at[sidx_a], rows_a, semg_a)

        @pl.when(jj == npair - 1)
        def _():
            pltpu.make_async_copy(rows_a, acc_sp.at[didx_t.at[j]],
                                  sems_a).wait()
        pltpu.make_async_copy(h_hbm.at[sidx_b], rows_b, semg_b).wait()
        pltpu.async_copy(rows_b, acc_sp.at[didx_t.at[j + 1]], sems_b,
                         add=True)

    pltpu.make_async_copy(rows_b, acc_sp.at[didx_t.at[2 * npair - 1]],
                          sems_b).wait()
    plsc.subcore_barrier()

    bufs = (rows_a, rows_b)
    sems = (sems_a, sems_b)

    def _o(r0):
        return pl.multiple_of(s * RPS + r0, 8)

    for i, (r0, rn) in enumerate(_OUT_CHUNKS):
        buf, sem = bufs[i % 2], sems[i % 2]
        if i >= 2:
            p0, pn = _OUT_CHUNKS[i - 2]
            pltpu.make_async_copy(buf, out_hbm.at[c, pl.ds(_o(p0), pn)],
                                  sem).wait()
        pltpu.sync_copy(acc_sp.at[pl.ds(_o(r0), rn)], buf)
        pltpu.async_copy(buf, out_hbm.at[c, pl.ds(_o(r0), rn)], sem)
    for i in (3, 4):
        r0, rn = _OUT_CHUNKS[i]
        pltpu.make_async_copy(bufs[i % 2],
                              out_hbm.at[c, pl.ds(_o(r0), rn)],
                              sems[i % 2]).wait()


_spmm_call = pl.kernel(
    _spmm_body,
    out_type=jax.ShapeDtypeStruct((NC, NPA, H), jnp.float32),
    mesh=_mesh,
    scratch_types=[
        pltpu.VMEM((CH,), jnp.int32),
        pltpu.VMEM((CH,), jnp.int32),
        pltpu.VMEM((CPW, CH), jnp.int32),
        pltpu.VMEM((CH, H), jnp.float32),
        pltpu.VMEM((CH, H), jnp.float32),
        pltpu.VMEM_SHARED((NPA, H), jnp.float32),
        pltpu.SemaphoreType.DMA,
        pltpu.SemaphoreType.DMA,
        pltpu.SemaphoreType.DMA,
        pltpu.SemaphoreType.DMA,
        pltpu.SemaphoreType.DMA,
    ],
)


def _silu(v):
    return v / (1.0 + jnp.exp(-v))


def _dinv_of(deg_ref):
    return lax.rsqrt(deg_ref[0] + deg_ref[1] + 1.0)


def _tcA_body(deg_ref, x_ref, w_ref, o_ref):
    dinv = _dinv_of(deg_ref)
    h = jnp.dot(x_ref[...], w_ref[...], preferred_element_type=jnp.float32)
    o_ref[...] = h * dinv[:, None]


def _tcB_body(deg_ref, acc_ref, hp_ref, b_ref, w_ref, o_ref):
    dinv = _dinv_of(deg_ref)
    y = (acc_ref[0] + acc_ref[1] + hp_ref[...]) * dinv[:, None] + b_ref[...]
    h1 = _silu(y)
    o_ref[...] = jnp.dot(h1, w_ref[...],
                         preferred_element_type=jnp.float32) * dinv[:, None]


def _tcC_body(deg_ref, acc_ref, hp_ref, b_ref, o_ref):
    dinv = _dinv_of(deg_ref)
    y = (acc_ref[0] + acc_ref[1] + hp_ref[...]) * dinv[:, None] + b_ref[...]
    o_ref[...] = _silu(y) * dinv[:, None]


def _tcD_body(deg_ref, acc_ref, hp_ref, bt_ref,
              muw_ref, mub_ref, lvw_ref, lvb_ref, eps_ref,
              d0w_ref, d0b_ref, d1w_ref, d1b_ref, mxw_ref, mxb_ref, lx_ref,
              omu_ref, olv_ref, omx_ref, olx_ref, qp_ref, cnt_ref):
    i = pl.program_id(0)

    @pl.when(i == 0)
    def _():
        qp_ref[...] = jnp.zeros_like(qp_ref)
        cnt_ref[...] = jnp.zeros_like(cnt_ref)

    dinv = _dinv_of(deg_ref)
    y3 = (acc_ref[0] + acc_ref[1] + hp_ref[...]) * dinv[:, None]
    gids = lax.broadcasted_iota(jnp.int32, (G, BR), 0)
    msk = (bt_ref[...] == gids).astype(jnp.float32)
    qp_ref[...] += jnp.dot(msk, y3, preferred_element_type=jnp.float32)
    cnt_ref[...] += jnp.sum(msk, axis=1)[None, :]

    @pl.when(i == NBLK - 1)
    def _():
        qp = qp_ref[...] / jnp.maximum(cnt_ref[0], 1.0)[:, None]
        mu = jnp.dot(qp, muw_ref[...],
                     preferred_element_type=jnp.float32) + mub_ref[...]
        lv = jnp.dot(qp, lvw_ref[...],
                     preferred_element_type=jnp.float32) + lvb_ref[...]
        z = mu + jnp.exp(0.5 * lv) * eps_ref[...]
        hd = jnp.tanh(jnp.dot(z, d0w_ref[...],
                              preferred_element_type=jnp.float32) + d0b_ref[...])
        hd = jnp.tanh(jnp.dot(hd, d1w_ref[...],
                              preferred_element_type=jnp.float32) + d1b_ref[...])
        mx = jnp.dot(hd, mxw_ref[...],
                     preferred_element_type=jnp.float32) + mxb_ref[...]
        omu_ref[...] = mu
        olv_ref[...] = lv
        omx_ref[...] = mx
        olx_ref[...] = jnp.broadcast_to(lx_ref[...], (G, OUT))


def _whole(shape):
    nd = len(shape)
    return pl.BlockSpec(shape, lambda i: (0,) * nd)


_deg_spec = pl.BlockSpec((2, BR), lambda i: (0, i))
_row_spec = pl.BlockSpec((BR, H), lambda i: (i, 0))
_acc_spec = pl.BlockSpec((2, BR, H), lambda i: (0, i, 0))

_tcA_call = pl.pallas_call(
    _tcA_body,
    grid=(NBLK,),
    in_specs=[_deg_spec, _row_spec, _whole((D, H))],
    out_specs=_row_spec,
    out_shape=jax.ShapeDtypeStruct((NPA, H), jnp.float32),
)

_tcB_call = pl.pallas_call(
    _tcB_body,
    grid=(NBLK,),
    in_specs=[_deg_spec, _acc_spec, _row_spec, _whole((1, H)), _whole((H, H))],
    out_specs=_row_spec,
    out_shape=jax.ShapeDtypeStruct((NPA, H), jnp.float32),
)

_tcC_call = pl.pallas_call(
    _tcC_body,
    grid=(NBLK,),
    in_specs=[_deg_spec, _acc_spec, _row_spec, _whole((1, H))],
    out_specs=_row_spec,
    out_shape=jax.ShapeDtypeStruct((NPA, H), jnp.float32),
)

_tcD_call = pl.pallas_call(
    _tcD_body,
    grid=(NBLK,),
    in_specs=[_deg_spec, _acc_spec, _row_spec,
              pl.BlockSpec((1, BR), lambda i: (0, i)),
              _whole((H, Z)), _whole((1, Z)), _whole((H, Z)), _whole((1, Z)),
              _whole((G, Z)),
              _whole((Z, DH)), _whole((1, DH)), _whole((DH, DH)),
              _whole((1, DH)), _whole((DH, OUT)), _whole((1, OUT)),
              _whole((1, OUT))],
    out_specs=[_whole((G, Z)), _whole((G, Z)), _whole((G, OUT)),
               _whole((G, OUT))],
    out_shape=[jax.ShapeDtypeStruct((G, Z), jnp.float32),
               jax.ShapeDtypeStruct((G, Z), jnp.float32),
               jax.ShapeDtypeStruct((G, OUT), jnp.float32),
               jax.ShapeDtypeStruct((G, OUT), jnp.float32)],
    scratch_shapes=[pltpu.VMEM((G, H), jnp.float32),
                    pltpu.VMEM((1, G), jnp.float32)],
)


def kernel(x, edge_index, batch, gc0_w, gc0_b, gc1_w, gc1_b, mu_w, mu_b,
           lv_w, lv_b, d0_w, d0_b, d1_w, d1_b, mx_w, mx_b, logvar_x_param):
    xp = jnp.pad(x, ((0, NPA - N), (0, 0)))
    bt = jnp.pad(batch, (0, NPA - N), constant_values=G).reshape(1, NPA)
    ei3 = edge_index.reshape(2, NCH, CH)
    eps = jax.random.normal(jax.random.key(42), (G, Z), jnp.float32)
    b0 = gc0_b.reshape(1, H)
    b1 = gc1_b.reshape(1, H)

    degs = _deg_call(ei3)
    h0p = _tcA_call(degs, xp, gc0_w)
    acc1 = _spmm_call(h0p, ei3)
    h1p = _tcB_call(degs, acc1, h0p, b0, gc1_w)
    acc2 = _spmm_call(h1p, ei3)
    h2p = _tcC_call(degs, acc2, h1p, b1)
    acc3 = _spmm_call(h2p, ei3)
    mu_zp, logvar_zp, mu_x, logvar_x = _tcD_call(
        degs, acc3, h2p, bt,
        mu_w, mu_b.reshape(1, Z), lv_w, lv_b.reshape(1, Z), eps,
        d0_w, d0_b.reshape(1, DH), d1_w, d1_b.reshape(1, DH),
        mx_w, mx_b.reshape(1, OUT), logvar_x_param.reshape(1, OUT))
    return (mu_zp, logvar_zp, mu_x, logvar_x)


# speedup vs baseline: 1.1136x; 1.0044x over previous
import jax
import jax.numpy as jnp
from jax import lax
from jax.experimental import pallas as pl
from jax.experimental.pallas import tpu as pltpu
from jax.experimental.pallas import tpu_sc as plsc

N = 10000
E = 320000
D = 128
H = 128
Z = 64
DH = 256
OUT = 231
G = 64

NC = 2
NS = 16
NW = NC * NS
CH = 128
NCH = E // CH
CPW = 80
NPA = 10240
RPS = NPA // NS
NPD = 10240
BR = 2560
NBLK = NPA // BR

_mesh = plsc.VectorSubcoreMesh(core_axis_name="c", subcore_axis_name="s")


def _zero_rows(buf, nrows):
    z = jnp.zeros((16,), jnp.float32)

    @pl.loop(0, nrows)
    def _(r):
        for k in range(8):
            buf[r, pl.ds(k * 16, 16)] = z


_OUT_CHUNKS = [(0, 128), (128, 128), (256, 128), (384, 128), (512, 128)]


def _stage_dst_table(ei_hbm, didx_t, wid, semi, wait):
    last = NW - 1

    @pl.when(wid < last)
    def _():
        off = pl.multiple_of(wid * CPW, 8)
        if wait:
            pltpu.make_async_copy(ei_hbm.at[1, pl.ds(off, CPW)], didx_t,
                                  semi).wait()
        else:
            pltpu.async_copy(ei_hbm.at[1, pl.ds(off, CPW)], didx_t, semi)

    @pl.when(wid == last)
    def _():
        ops = [(ei_hbm.at[1, pl.ds(NCH - 20, 16)], didx_t.at[pl.ds(0, 16)])]
        ops += [(ei_hbm.at[1, NCH - 4 + k], didx_t.at[16 + k])
                for k in range(4)]
        for src, dst in ops:
            if wait:
                pltpu.make_async_copy(src, dst, semi).wait()
            else:
                pltpu.async_copy(src, dst, semi)


def _deg_body(ei_hbm, deg_out, didx_t, ones_v, zrow_v, deg_sp, semi, semd):
    c = lax.axis_index("c")
    s = lax.axis_index("s")
    wid = s * NC + c
    cb = jnp.minimum(wid * CPW, NCH - 20)
    nch = jnp.where(wid == NW - 1, 20, CPW)

    _stage_dst_table(ei_hbm, didx_t, wid, semi, wait=False)

    @pl.loop(0, NPD // NS // 16)
    def _(r):
        zrow_v[pl.ds(r * 16, 16)] = jnp.zeros((16,), jnp.float32)
    for k in range(8):
        ones_v[pl.ds(k * 16, 16)] = jnp.ones((16,), jnp.float32)
    pltpu.sync_copy(zrow_v,
                    deg_sp.at[pl.ds(pl.multiple_of(s * (NPD // NS), 8),
                                    NPD // NS)])
    _stage_dst_table(ei_hbm, didx_t, wid, semi, wait=True)
    plsc.subcore_barrier()

    @pl.loop(0, nch // 2)
    def _(jj):
        j = jj * 2
        pltpu.async_copy(ones_v, deg_sp.at[didx_t.at[j]], semi, add=True)
        pltpu.async_copy(ones_v, deg_sp.at[didx_t.at[j + 1]], semd, add=True)
        pltpu.make_async_copy(ones_v, deg_sp.at[didx_t.at[j]], semi).wait()
        pltpu.make_async_copy(ones_v, deg_sp.at[didx_t.at[j + 1]],
                              semd).wait()

    plsc.subcore_barrier()
    pltpu.sync_copy(
        deg_sp.at[pl.ds(pl.multiple_of(s * (NPD // NS), 8), NPD // NS)],
        zrow_v)
    pltpu.sync_copy(
        zrow_v,
        deg_out.at[c, pl.ds(pl.multiple_of(s * (NPD // NS), 8), NPD // NS)])


_deg_call = pl.kernel(
    _deg_body,
    out_type=jax.ShapeDtypeStruct((NC, NPD), jnp.float32),
    mesh=_mesh,
    scratch_types=[
        pltpu.VMEM((CPW, CH), jnp.int32),
        pltpu.VMEM((CH,), jnp.float32),
        pltpu.VMEM((NPD // NS,), jnp.float32),
        pltpu.VMEM_SHARED((NPD,), jnp.float32),
        pltpu.SemaphoreType.DMA,
        pltpu.SemaphoreType.DMA,
    ],
)


def _spmm_body(h_hbm, ei_hbm, out_hbm,
               sidx_a, sidx_b, didx_t, rows_a, rows_b, acc_sp,
               semg_a, semg_b, sems_a, sems_b, semi):
    c = lax.axis_index("c")
    s = lax.axis_index("s")
    wid = s * NC + c
    cb = jnp.minimum(wid * CPW, NCH - 20)
    npair = jnp.where(wid == NW - 1, 10, CPW // 2)

    _stage_dst_table(ei_hbm, didx_t, wid, semi, wait=False)
    pltpu.sync_copy(ei_hbm.at[0, cb], sidx_a)
    pltpu.async_copy(h_hbm.at[sidx_a], rows_a, semg_a)

    _zero_rows(rows_b, CH)
    for (r0, rn) in _OUT_CHUNKS:
        off = pl.multiple_of(s * RPS + r0, 8)
        pltpu.async_copy(rows_b, acc_sp.at[pl.ds(off, rn)], sems_b)
    for (r0, rn) in _OUT_CHUNKS:
        off = pl.multiple_of(s * RPS + r0, 8)
        pltpu.make_async_copy(rows_b, acc_sp.at[pl.ds(off, rn)],
                              sems_b).wait()
    _stage_dst_table(ei_hbm, didx_t, wid, semi, wait=True)
    plsc.subcore_barrier()

    @pl.loop(0, npair)
    def _(jj):
        j = jj * 2
        pltpu.sync_copy(ei_hbm.at[0, cb + j + 1], sidx_b)

        @pl.when(jj > 0)
        def _():
            pltpu.make_async_copy(rows_b, acc_sp.at[didx_t.at[j - 1]],
                                  sems_b).wait()
        pltpu.async_copy(h_hbm.at[sidx_b], rows_b, semg_b)
        pltpu.make_async_copy(h_hbm.at[sidx_a], rows_a, semg_a).wait()
        pltpu.async_copy(rows_a, acc_sp.at[didx_t.at[j]], sems_a,
                         add=True)

        @pl.when(jj < npair - 1)
        def _():
            pltpu.sync_copy(ei_hbm.at[0, cb + j + 2], sidx_a)
            pltpu.make_async_copy(rows_a, acc_sp.at[didx_t.at[j]],
                                  sems_a).wait()
            pltpu.async_copy(h_hbm.at[sidx_a], rows_a, semg_a)

        @pl.when(jj == npair - 1)
        def _():
            pltpu.make_async_copy(rows_a, acc_sp.at[didx_t.at[j]],
                                  sems_a).wait()
        pltpu.make_async_copy(h_hbm.at[sidx_b], rows_b, semg_b).wait()
        pltpu.async_copy(rows_b, acc_sp.at[didx_t.at[j + 1]], sems_b,
                         add=True)

    pltpu.make_async_copy(rows_b, acc_sp.at[didx_t.at[2 * npair - 1]],
                          sems_b).wait()
    plsc.subcore_barrier()

    bufs = (rows_a, rows_b)
    sems = (sems_a, sems_b)

    def _o(r0):
        return pl.multiple_of(s * RPS + r0, 8)

    for i, (r0, rn) in enumerate(_OUT_CHUNKS):
        buf, sem = bufs[i % 2], sems[i % 2]
        if i >= 2:
            p0, pn = _OUT_CHUNKS[i - 2]
            pltpu.make_async_copy(buf, out_hbm.at[c, pl.ds(_o(p0), pn)],
                                  sem).wait()
        pltpu.sync_copy(acc_sp.at[pl.ds(_o(r0), rn)], buf)
        pltpu.async_copy(buf, out_hbm.at[c, pl.ds(_o(r0), rn)], sem)
    for i in (3, 4):
        r0, rn = _OUT_CHUNKS[i]
        pltpu.make_async_copy(bufs[i % 2],
                              out_hbm.at[c, pl.ds(_o(r0), rn)],
                              sems[i % 2]).wait()


_spmm_call = pl.kernel(
    _spmm_body,
    out_type=jax.ShapeDtypeStruct((NC, NPA, H), jnp.float32),
    mesh=_mesh,
    scratch_types=[
        pltpu.VMEM((CH,), jnp.int32),
        pltpu.VMEM((CH,), jnp.int32),
        pltpu.VMEM((CPW, CH), jnp.int32),
        pltpu.VMEM((CH, H), jnp.float32),
        pltpu.VMEM((CH, H), jnp.float32),
        pltpu.VMEM_SHARED((NPA, H), jnp.float32),
        pltpu.SemaphoreType.DMA,
        pltpu.SemaphoreType.DMA,
        pltpu.SemaphoreType.DMA,
        pltpu.SemaphoreType.DMA,
        pltpu.SemaphoreType.DMA,
    ],
)


def _silu(v):
    return v / (1.0 + jnp.exp(-v))


def _dinv_of(deg_ref):
    return lax.rsqrt(deg_ref[0] + deg_ref[1] + 1.0)


def _tcA_body(deg_ref, x_ref, w_ref, o_ref):
    dinv = _dinv_of(deg_ref)
    h = jnp.dot(x_ref[...], w_ref[...], preferred_element_type=jnp.float32)
    o_ref[...] = h * dinv[:, None]


def _tcB_body(deg_ref, acc_ref, hp_ref, b_ref, w_ref, o_ref):
    dinv = _dinv_of(deg_ref)
    y = (acc_ref[0] + acc_ref[1] + hp_ref[...]) * dinv[:, None] + b_ref[...]
    h1 = _silu(y)
    o_ref[...] = jnp.dot(h1, w_ref[...],
                         preferred_element_type=jnp.float32) * dinv[:, None]


def _tcC_body(deg_ref, acc_ref, hp_ref, b_ref, o_ref):
    dinv = _dinv_of(deg_ref)
    y = (acc_ref[0] + acc_ref[1] + hp_ref[...]) * dinv[:, None] + b_ref[...]
    o_ref[...] = _silu(y) * dinv[:, None]


def _tcD_body(deg_ref, acc_ref, hp_ref, bt_ref,
              muw_ref, mub_ref, lvw_ref, lvb_ref, eps_ref,
              d0w_ref, d0b_ref, d1w_ref, d1b_ref, mxw_ref, mxb_ref, lx_ref,
              omu_ref, olv_ref, omx_ref, olx_ref, qp_ref, cnt_ref):
    i = pl.program_id(0)

    @pl.when(i == 0)
    def _():
        qp_ref[...] = jnp.zeros_like(qp_ref)
        cnt_ref[...] = jnp.zeros_like(cnt_ref)

    dinv = _dinv_of(deg_ref)
    y3 = (acc_ref[0] + acc_ref[1] + hp_ref[...]) * dinv[:, None]
    gids = lax.broadcasted_iota(jnp.int32, (G, BR), 0)
    msk = (bt_ref[...] == gids).astype(jnp.float32)
    qp_ref[...] += jnp.dot(msk, y3, preferred_element_type=jnp.float32)
    cnt_ref[...] += jnp.sum(msk, axis=1)[None, :]

    @pl.when(i == NBLK - 1)
    def _():
        qp = qp_ref[...] / jnp.maximum(cnt_ref[0], 1.0)[:, None]
        mu = jnp.dot(qp, muw_ref[...],
                     preferred_element_type=jnp.float32) + mub_ref[...]
        lv = jnp.dot(qp, lvw_ref[...],
                     preferred_element_type=jnp.float32) + lvb_ref[...]
        z = mu + jnp.exp(0.5 * lv) * eps_ref[...]
        hd = jnp.tanh(jnp.dot(z, d0w_ref[...],
                              preferred_element_type=jnp.float32) + d0b_ref[...])
        hd = jnp.tanh(jnp.dot(hd, d1w_ref[...],
                              preferred_element_type=jnp.float32) + d1b_ref[...])
        mx = jnp.dot(hd, mxw_ref[...],
                     preferred_element_type=jnp.float32) + mxb_ref[...]
        omu_ref[...] = mu
        olv_ref[...] = lv
        omx_ref[...] = mx
        olx_ref[...] = jnp.broadcast_to(lx_ref[...], (G, OUT))


def _whole(shape):
    nd = len(shape)
    return pl.BlockSpec(shape, lambda i: (0,) * nd)


_deg_spec = pl.BlockSpec((2, BR), lambda i: (0, i))
_row_spec = pl.BlockSpec((BR, H), lambda i: (i, 0))
_acc_spec = pl.BlockSpec((2, BR, H), lambda i: (0, i, 0))

_tcA_call = pl.pallas_call(
    _tcA_body,
    grid=(NBLK,),
    in_specs=[_deg_spec, _row_spec, _whole((D, H))],
    out_specs=_row_spec,
    out_shape=jax.ShapeDtypeStruct((NPA, H), jnp.float32),
)

_tcB_call = pl.pallas_call(
    _tcB_body,
    grid=(NBLK,),
    in_specs=[_deg_spec, _acc_spec, _row_spec, _whole((1, H)), _whole((H, H))],
    out_specs=_row_spec,
    out_shape=jax.ShapeDtypeStruct((NPA, H), jnp.float32),
)

_tcC_call = pl.pallas_call(
    _tcC_body,
    grid=(NBLK,),
    in_specs=[_deg_spec, _acc_spec, _row_spec, _whole((1, H))],
    out_specs=_row_spec,
    out_shape=jax.ShapeDtypeStruct((NPA, H), jnp.float32),
)

_tcD_call = pl.pallas_call(
    _tcD_body,
    grid=(NBLK,),
    in_specs=[_deg_spec, _acc_spec, _row_spec,
              pl.BlockSpec((1, BR), lambda i: (0, i)),
              _whole((H, Z)), _whole((1, Z)), _whole((H, Z)), _whole((1, Z)),
              _whole((G, Z)),
              _whole((Z, DH)), _whole((1, DH)), _whole((DH, DH)),
              _whole((1, DH)), _whole((DH, OUT)), _whole((1, OUT)),
              _whole((1, OUT))],
    out_specs=[_whole((G, Z)), _whole((G, Z)), _whole((G, OUT)),
               _whole((G, OUT))],
    out_shape=[jax.ShapeDtypeStruct((G, Z), jnp.float32),
               jax.ShapeDtypeStruct((G, Z), jnp.float32),
               jax.ShapeDtypeStruct((G, OUT), jnp.float32),
               jax.ShapeDtypeStruct((G, OUT), jnp.float32)],
    scratch_shapes=[pltpu.VMEM((G, H), jnp.float32),
                    pltpu.VMEM((1, G), jnp.float32)],
)


def kernel(x, edge_index, batch, gc0_w, gc0_b, gc1_w, gc1_b, mu_w, mu_b,
           lv_w, lv_b, d0_w, d0_b, d1_w, d1_b, mx_w, mx_b, logvar_x_param):
    xp = jnp.pad(x, ((0, NPA - N), (0, 0)))
    bt = jnp.pad(batch, (0, NPA - N), constant_values=G).reshape(1, NPA)
    ei3 = edge_index.reshape(2, NCH, CH)
    eps = jax.random.normal(jax.random.key(42), (G, Z), jnp.float32)
    b0 = gc0_b.reshape(1, H)
    b1 = gc1_b.reshape(1, H)

    degs = _deg_call(ei3)
    h0p = _tcA_call(degs, xp, gc0_w)
    acc1 = _spmm_call(h0p, ei3)
    h1p = _tcB_call(degs, acc1, h0p, b0, gc1_w)
    acc2 = _spmm_call(h1p, ei3)
    h2p = _tcC_call(degs, acc2, h1p, b1)
    acc3 = _spmm_call(h2p, ei3)
    mu_zp, logvar_zp, mu_x, logvar_x = _tcD_call(
        degs, acc3, h2p, bt,
        mu_w, mu_b.reshape(1, Z), lv_w, lv_b.reshape(1, Z), eps,
        d0_w, d0_b.reshape(1, DH), d1_w, d1_b.reshape(1, DH),
        mx_w, mx_b.reshape(1, OUT), logvar_x_param.reshape(1, OUT))
    return (mu_zp, logvar_zp, mu_x, logvar_x)
